# baseline (speedup 1.0000x reference)
import functools

import jax
import jax.numpy as jnp
from jax import lax
from jax.experimental import pallas as pl
from jax.experimental.pallas import tpu as pltpu
from jax.experimental.pallas import tpu_sc as plsc

_BATCH, _NT, _D = 4, 8192, 768
_NC, _NS = 2, 16
_NW = _NC * _NS
_TPW = _NT // _NW
_CHUNK = 64


def _sc_body(tok_hbm, tab_hbm, out_hbm, tok_v, tab_v):
    wid = lax.axis_index("s") * _NC + lax.axis_index("c")
    t0 = wid * _TPW

    def chunk_body(ci, _):
        base = t0 + ci * _CHUNK
        pltpu.sync_copy(tab_hbm.at[pl.ds(base, _CHUNK)], tab_v)
        for b in range(_BATCH):
            pltpu.sync_copy(tok_hbm.at[b, pl.ds(base, _CHUNK)], tok_v)

            def add_row(r, _):
                for c in range(_D // 16):
                    o = c * 16
                    tok_v[r, pl.ds(o, 16)] = (
                        tok_v[r, pl.ds(o, 16)] + tab_v[r, pl.ds(o, 16)]
                    )
                return 0

            lax.fori_loop(0, _CHUNK, add_row, 0)
            pltpu.sync_copy(tok_v, out_hbm.at[b, pl.ds(base, _CHUNK)])
        return 0

    lax.fori_loop(0, _TPW // _CHUNK, chunk_body, 0)


def kernel(encoded_tokens, position_table):
    mesh = plsc.VectorSubcoreMesh(core_axis_name="c", subcore_axis_name="s")
    run = functools.partial(
        pl.kernel,
        mesh=mesh,
        out_type=jax.ShapeDtypeStruct((_BATCH, _NT, _D), jnp.float32),
        scratch_types=[
            pltpu.VMEM((_CHUNK, _D), jnp.float32),
            pltpu.VMEM((_CHUNK, _D), jnp.float32),
        ],
    )(_sc_body)
    return run(encoded_tokens, position_table)


# speedup vs baseline: 1.3291x; 1.3291x over previous
import functools

import jax
import jax.numpy as jnp
from jax import lax
from jax.experimental import pallas as pl
from jax.experimental.pallas import tpu as pltpu
from jax.experimental.pallas import tpu_sc as plsc

_BATCH, _NT, _D = 4, 8192, 768
_NC, _NS = 2, 16
_NW = _NC * _NS
_TPW = _NT // _NW
_CHUNK = 32
_NCH = _TPW // _CHUNK


def _sc_body(tok_hbm, tab_hbm, out_hbm,
             tok0, tok1, tab0, tab1,
             l0, l1, s0, s1, t0sem, t1sem):
    wid = lax.axis_index("s") * _NC + lax.axis_index("c")
    t0 = wid * _TPW

    toks = [tok0, tok1]
    tabs = [tab0, tab1]
    lsems = [l0, l1]
    ssems = [s0, s1]
    tsems = [t0sem, t1sem]

    def tab_load(ci):
        base = t0 + ci * _CHUNK
        return pltpu.async_copy(
            tab_hbm.at[pl.ds(base, _CHUNK)], tabs[ci % 2], tsems[ci % 2]
        )

    def tok_load(ci, b, k):
        base = t0 + ci * _CHUNK
        return pltpu.async_copy(
            tok_hbm.at[b, pl.ds(base, _CHUNK)], toks[k], lsems[k]
        )

    def tok_store(ci, b, k):
        base = t0 + ci * _CHUNK
        return pltpu.async_copy(
            toks[k], out_hbm.at[b, pl.ds(base, _CHUNK)], ssems[k]
        )

    nsteps = _NCH * _BATCH
    tab_h = [None] * _NCH
    load_h = [None] * nsteps
    store_h = [None] * nsteps

    tab_h[0] = tab_load(0)
    load_h[0] = tok_load(0, 0, 0)

    for s in range(nsteps):
        ci, b, k = s // _BATCH, s % _BATCH, s % 2
        if b == 0 and ci + 1 < _NCH:
            tab_h[ci + 1] = tab_load(ci + 1)
        if s + 1 < nsteps:
            if s >= 1:
                store_h[s - 1].wait()
            nci, nb = (s + 1) // _BATCH, (s + 1) % _BATCH
            load_h[s + 1] = tok_load(nci, nb, (s + 1) % 2)
        if b == 0:
            tab_h[ci].wait()
        load_h[s].wait()

        tok_v, tab_v = toks[k], tabs[ci % 2]

        def add_row(r, _):
            for c in range(_D // 16):
                o = c * 16
                tok_v[r, pl.ds(o, 16)] = (
                    tok_v[r, pl.ds(o, 16)] + tab_v[r, pl.ds(o, 16)]
                )
            return 0

        lax.fori_loop(0, _CHUNK, add_row, 0)
        store_h[s] = tok_store(ci, b, k)

    store_h[nsteps - 2].wait()
    store_h[nsteps - 1].wait()


def kernel(encoded_tokens, position_table):
    mesh = plsc.VectorSubcoreMesh(core_axis_name="c", subcore_axis_name="s")
    run = functools.partial(
        pl.kernel,
        mesh=mesh,
        out_type=jax.ShapeDtypeStruct((_BATCH, _NT, _D), jnp.float32),
        scratch_types=[
            pltpu.VMEM((_CHUNK, _D), jnp.float32),
            pltpu.VMEM((_CHUNK, _D), jnp.float32),
            pltpu.VMEM((_CHUNK, _D), jnp.float32),
            pltpu.VMEM((_CHUNK, _D), jnp.float32),
            pltpu.SemaphoreType.DMA,
            pltpu.SemaphoreType.DMA,
            pltpu.SemaphoreType.DMA,
            pltpu.SemaphoreType.DMA,
            pltpu.SemaphoreType.DMA,
            pltpu.SemaphoreType.DMA,
        ],
    )(_sc_body)
    return run(encoded_tokens, position_table)


# speedup vs baseline: 1.3455x; 1.0123x over previous
import functools

import jax
import jax.numpy as jnp
from jax import lax
from jax.experimental import pallas as pl
from jax.experimental.pallas import tpu as pltpu
from jax.experimental.pallas import tpu_sc as plsc

_BATCH, _NT, _D = 4, 8192, 768
_NC, _NS = 2, 16
_NW = _NC * _NS
_TPW = _NT // _NW
_CHUNK = 32
_NCH = _TPW // _CHUNK
_NBUF = 3


def _sc_body(tok_hbm, tab_hbm, out_hbm,
             tok0, tok1, tok2, tab0, tab1,
             l0, l1, l2, s0, s1, s2, tsa, tsb):
    wid = lax.axis_index("s") * _NC + lax.axis_index("c")
    t0 = wid * _TPW

    toks = [tok0, tok1, tok2]
    tabs = [tab0, tab1]
    lsems = [l0, l1, l2]
    ssems = [s0, s1, s2]
    tsems = [tsa, tsb]

    def tab_load(ci):
        base = t0 + ci * _CHUNK
        return pltpu.async_copy(
            tab_hbm.at[pl.ds(base, _CHUNK)], tabs[ci % 2], tsems[ci % 2]
        )

    def tok_load(s):
        ci, b, k = s // _BATCH, s % _BATCH, s % _NBUF
        base = t0 + ci * _CHUNK
        return pltpu.async_copy(
            tok_hbm.at[b, pl.ds(base, _CHUNK)], toks[k], lsems[k]
        )

    def tok_store(s):
        ci, b, k = s // _BATCH, s % _BATCH, s % _NBUF
        base = t0 + ci * _CHUNK
        return pltpu.async_copy(
            toks[k], out_hbm.at[b, pl.ds(base, _CHUNK)], ssems[k]
        )

    nsteps = _NCH * _BATCH
    tab_h = [None] * _NCH
    load_h = [None] * nsteps
    store_h = [None] * nsteps

    tab_h[0] = tab_load(0)
    load_h[0] = tok_load(0)
    load_h[1] = tok_load(1)

    for s in range(nsteps):
        ci, b, k = s // _BATCH, s % _BATCH, s % _NBUF
        if b == 0 and ci + 1 < _NCH:
            tab_h[ci + 1] = tab_load(ci + 1)
        if b == 0:
            tab_h[ci].wait()
        load_h[s].wait()

        tok_v, tab_v = toks[k], tabs[ci % 2]

        def add_row(r, _):
            for c in range(_D // 16):
                o = c * 16
                plsc.addupdate(tok_v.at[r, pl.ds(o, 16)], tab_v[r, pl.ds(o, 16)])
            return 0

        lax.fori_loop(0, _CHUNK, add_row, 0)
        store_h[s] = tok_store(s)

        if s + 2 < nsteps:
            if s >= 1:
                store_h[s - 1].wait()
            load_h[s + 2] = tok_load(s + 2)

    store_h[nsteps - 3].wait()
    store_h[nsteps - 2].wait()
    store_h[nsteps - 1].wait()


def kernel(encoded_tokens, position_table):
    mesh = plsc.VectorSubcoreMesh(core_axis_name="c", subcore_axis_name="s")
    run = functools.partial(
        pl.kernel,
        mesh=mesh,
        out_type=jax.ShapeDtypeStruct((_BATCH, _NT, _D), jnp.float32),
        scratch_types=[
            pltpu.VMEM((_CHUNK, _D), jnp.float32),
            pltpu.VMEM((_CHUNK, _D), jnp.float32),
            pltpu.VMEM((_CHUNK, _D), jnp.float32),
            pltpu.VMEM((_CHUNK, _D), jnp.float32),
            pltpu.VMEM((_CHUNK, _D), jnp.float32),
            pltpu.SemaphoreType.DMA,
            pltpu.SemaphoreType.DMA,
            pltpu.SemaphoreType.DMA,
            pltpu.SemaphoreType.DMA,
            pltpu.SemaphoreType.DMA,
            pltpu.SemaphoreType.DMA,
            pltpu.SemaphoreType.DMA,
            pltpu.SemaphoreType.DMA,
        ],
    )(_sc_body)
    return run(encoded_tokens, position_table)


# speedup vs baseline: 1.6001x; 1.1892x over previous
import functools

import jax
import jax.numpy as jnp
from jax import lax
from jax.experimental import pallas as pl
from jax.experimental.pallas import tpu as pltpu
from jax.experimental.pallas import tpu_sc as plsc

_BATCH, _NT, _D = 4, 8192, 768
_NC, _NS = 2, 16
_NW = _NC * _NS
_TPW = _NT // _NW
_CHUNK = 16
_NCH = _TPW // _CHUNK


def _sc_body(tok_hbm, tab_hbm, out_hbm, *refs):
    toks = list(refs[0:8])
    tabs = list(refs[8:10])
    lsems = list(refs[10:18])
    ssems = list(refs[18:26])
    tsems = list(refs[26:28])

    wid = lax.axis_index("s") * _NC + lax.axis_index("c")
    t0 = wid * _TPW

    def tab_copy(ci, par):
        return pltpu.make_async_copy(
            tab_hbm.at[pl.ds(t0 + ci * _CHUNK, _CHUNK)], tabs[par], tsems[par]
        )

    def tok_copy(ci, b, slot):
        return pltpu.make_async_copy(
            tok_hbm.at[b, pl.ds(t0 + ci * _CHUNK, _CHUNK)],
            toks[slot], lsems[slot],
        )

    def out_copy(ci, b, slot):
        return pltpu.make_async_copy(
            toks[slot], out_hbm.at[b, pl.ds(t0 + ci * _CHUNK, _CHUNK)],
            ssems[slot],
        )

    tab_copy(0, 0).start()
    for b in range(_BATCH):
        tok_copy(0, b, b).start()

    def chunk_iter(ci, _):
        def body(par):
            cur = par * _BATCH
            nxt = (1 - par) * _BATCH
            tab_copy(ci, par).wait()

            @pl.when(ci + 1 < _NCH)
            def _():
                tab_copy(ci + 1, 1 - par).start()

            tab_v = tabs[par]
            for b in range(_BATCH):
                slot = cur + b
                tok_copy(ci, b, slot).wait()
                tok_v = toks[slot]

                def add_row(r, _):
                    for c in range(_D // 16):
                        o = c * 16
                        plsc.addupdate(
                            tok_v.at[r, pl.ds(o, 16)], tab_v[r, pl.ds(o, 16)]
                        )
                    return 0

                lax.fori_loop(0, _CHUNK, add_row, 0)
                out_copy(ci, b, slot).start()

                @pl.when(ci > 0)
                def _():
                    out_copy(ci - 1, b, nxt + b).wait()

                @pl.when(ci + 1 < _NCH)
                def _():
                    tok_copy(ci + 1, b, nxt + b).start()

        lax.cond(ci % 2 == 0, lambda: body(0), lambda: body(1))
        return 0

    lax.fori_loop(0, _NCH, chunk_iter, 0)

    for b in range(_BATCH):
        out_copy(_NCH - 1, b, _BATCH + b).wait()


def kernel(encoded_tokens, position_table):
    mesh = plsc.VectorSubcoreMesh(core_axis_name="c", subcore_axis_name="s")
    scratch = (
        [pltpu.VMEM((_CHUNK, _D), jnp.float32)] * 8
        + [pltpu.VMEM((_CHUNK, _D), jnp.float32)] * 2
        + [pltpu.SemaphoreType.DMA] * 18
    )
    run = functools.partial(
        pl.kernel,
        mesh=mesh,
        out_type=jax.ShapeDtypeStruct((_BATCH, _NT, _D), jnp.float32),
        scratch_types=scratch,
    )(_sc_body)
    return run(encoded_tokens, position_table)


# speedup vs baseline: 1.6079x; 1.0049x over previous
import functools

import jax
import jax.numpy as jnp
from jax import lax
from jax.experimental import pallas as pl
from jax.experimental.pallas import tpu as pltpu
from jax.experimental.pallas import tpu_sc as plsc

_BATCH, _NT, _D = 4, 8192, 768
_NC, _NS = 2, 16
_NW = _NC * _NS
_TPW = _NT // _NW
_CHUNK = 16
_NCH = _TPW // _CHUNK


def _sc_body(tok_hbm, tab_hbm, out_hbm, *refs):
    toks = list(refs[0:8])
    tabs = list(refs[8:10])
    lsems = list(refs[10:18])
    ssems = list(refs[18:26])
    tsems = list(refs[26:28])

    wid = lax.axis_index("s") * _NC + lax.axis_index("c")
    t0 = wid * _TPW

    def tab_copy(ci, par):
        return pltpu.make_async_copy(
            tab_hbm.at[pl.ds(t0 + ci * _CHUNK, _CHUNK)], tabs[par], tsems[par]
        )

    def tok_copy(ci, b, slot):
        return pltpu.make_async_copy(
            tok_hbm.at[b, pl.ds(t0 + ci * _CHUNK, _CHUNK)],
            toks[slot], lsems[slot],
        )

    def out_copy(ci, b, slot):
        return pltpu.make_async_copy(
            toks[slot], out_hbm.at[b, pl.ds(t0 + ci * _CHUNK, _CHUNK)],
            ssems[slot],
        )

    tab_copy(0, 0).start()
    for b in range(_BATCH):
        tok_copy(0, b, b).start()

    def chunk_iter(ci, _):
        def body(par):
            cur = par * _BATCH
            nxt = (1 - par) * _BATCH
            tab_copy(ci, par).wait()

            @pl.when(ci + 1 < _NCH)
            def _():
                tab_copy(ci + 1, 1 - par).start()

            tab_v = tabs[par]
            for b in range(_BATCH):
                slot = cur + b

                @pl.when(ci > 0)
                def _():
                    out_copy(ci - 1, b, nxt + b).wait()

                @pl.when(ci + 1 < _NCH)
                def _():
                    tok_copy(ci + 1, b, nxt + b).start()

                tok_copy(ci, b, slot).wait()
                tok_v = toks[slot]

                def add_row(r, _):
                    for c in range(_D // 16):
                        o = c * 16
                        plsc.addupdate(
                            tok_v.at[r, pl.ds(o, 16)], tab_v[r, pl.ds(o, 16)]
                        )
                    return 0

                lax.fori_loop(0, _CHUNK, add_row, 0)
                out_copy(ci, b, slot).start()

        lax.cond(ci % 2 == 0, lambda: body(0), lambda: body(1))
        return 0

    lax.fori_loop(0, _NCH, chunk_iter, 0)

    for b in range(_BATCH):
        out_copy(_NCH - 1, b, _BATCH + b).wait()


def kernel(encoded_tokens, position_table):
    mesh = plsc.VectorSubcoreMesh(core_axis_name="c", subcore_axis_name="s")
    scratch = (
        [pltpu.VMEM((_CHUNK, _D), jnp.float32)] * 8
        + [pltpu.VMEM((_CHUNK, _D), jnp.float32)] * 2
        + [pltpu.SemaphoreType.DMA] * 18
    )
    run = functools.partial(
        pl.kernel,
        mesh=mesh,
        out_type=jax.ShapeDtypeStruct((_BATCH, _NT, _D), jnp.float32),
        scratch_types=scratch,
    )(_sc_body)
    return run(encoded_tokens, position_table)
